# baseline (speedup 1.0000x reference)
import functools

import jax
import jax.numpy as jnp
from jax import lax
from jax.experimental import pallas as pl
from jax.experimental.pallas import tpu as pltpu
from jax.experimental.pallas import tpu_sc as plsc

NUM_CORES = 2
NUM_SUBCORES = 16
NUM_WORKERS = NUM_CORES * NUM_SUBCORES


@functools.cache
def _make_min_kernel(batch):
    bpw = batch // NUM_WORKERS
    mesh = plsc.VectorSubcoreMesh(core_axis_name="c", subcore_axis_name="s")

    @functools.partial(
        pl.kernel,
        out_type=jax.ShapeDtypeStruct((batch,), jnp.float32),
        mesh=mesh,
        compiler_params=pltpu.CompilerParams(
            needs_layout_passes=False, use_tc_tiling_on_sc=True
        ),
        scratch_types=[
            pltpu.VMEM((bpw,), jnp.float32),
            pltpu.SemaphoreType.DMA,
        ],
    )
    def sc_kernel(movies_hbm, users_hbm, mtab_hbm, out_hbm, v, sem):
        wid = lax.axis_index("s") * NUM_CORES + lax.axis_index("c")
        base = wid * bpw
        def zero_body(g, _):
            v[pl.ds(g * 16, 16)] = jnp.zeros((16,), jnp.float32)
            return 0

        lax.fori_loop(0, bpw // 16, zero_body, 0)
        pltpu.sync_copy(v, out_hbm.at[pl.ds(base, bpw)])

    return jax.jit(sc_kernel)


def kernel(movies, users, movie_table, user_table):
    batch = movies.shape[0]
    out = _make_min_kernel(batch)(
        movies.astype(jnp.int32), users.astype(jnp.int32), movie_table
    )
    return out.reshape(batch, 1)
